# Initial kernel scaffold; baseline (speedup 1.0000x reference)
import jax
import jax.numpy as jnp
from jax.experimental import pallas as pl


def kernel(x, edge_index, Wl1, Wr1, b1, gamma, beta, rm, rv, Wl2, Wr2, b2):
    raise NotImplementedError("write your pallas kernel here")


# speedup vs baseline: 5.6487x; 5.6487x over previous
import functools

import jax
import jax.numpy as jnp
from jax import lax
from jax.experimental import pallas as pl
from jax.experimental.pallas import tpu as pltpu
from jax.experimental.pallas import tpu_sc as plsc

N = 10000
E = 320000
NFEAT = 128
NHID = 128
NCLASS = 47
EPS = 1e-5

NC = 2
NS = 16
NW = NC * NS
B = 128
NCHUNK1 = 158
NCHUNK2 = 79
EPAD = NS * NCHUNK1 * B
WH = 80
W2 = 48
RPT = 632
NACC = NS * RPT

BN = 1000
GRID = N // BN


def _zero_fill(zbuf, width):
  zero16 = jnp.zeros((16,), jnp.float32)
  def zrow(i, _):
    for j in range(width // 16):
      zbuf[i, pl.ds(j * 16, 16)] = zero16
    return 0
  lax.fori_loop(0, B, zrow, 0)


def _blk_copy(src, dst):
  for k in range(RPT // B):
    pltpu.sync_copy(src(k * B, B), dst(k * B, B))
  rem = RPT % B
  if rem:
    pltpu.sync_copy(src((RPT // B) * B, rem), dst((RPT // B) * B, rem))


_MESH = plsc.VectorSubcoreMesh(core_axis_name="c", subcore_axis_name="s")


@functools.partial(
    pl.kernel,
    out_type=jax.ShapeDtypeStruct((NC, NACC, WH), jnp.float32),
    mesh=_MESH,
    compiler_params=pltpu.CompilerParams(use_tc_tiling_on_sc=False),
    scratch_types=[
        pltpu.VMEM((NCHUNK1, B), jnp.int32),
        pltpu.VMEM((NCHUNK1, B), jnp.int32),
        pltpu.VMEM((B, WH), jnp.float32),
        pltpu.VMEM((B, WH), jnp.float32),
        pltpu.VMEM_SHARED((NACC, WH), jnp.float32),
        pltpu.SemaphoreType.DMA,
    ],
)
def _sc_agg1(table_hbm, sidx_hbm, didx_hbm, out_hbm,
             sidx_v, didx_v, gbuf, zbuf, acc_sh, sem):
  cid = lax.axis_index("c")
  sid = lax.axis_index("s")

  _zero_fill(zbuf, WH)
  base = sid * RPT
  _blk_copy(lambda o, n: zbuf.at[pl.ds(0, n)],
            lambda o, n: acc_sh.at[pl.ds(base + o, n)])

  pltpu.sync_copy(sidx_hbm.at[sid], sidx_v)
  pltpu.sync_copy(didx_hbm.at[sid], didx_v)
  plsc.subcore_barrier()

  my_table = table_hbm.at[cid]
  def chunk(j, _):
    pltpu.async_copy(my_table.at[sidx_v.at[j]], gbuf, sem).wait()
    pltpu.sync_copy(gbuf, acc_sh.at[didx_v.at[j]], add=True)
    return 0
  lax.fori_loop(0, NCHUNK1, chunk, 0)
  plsc.subcore_barrier()

  _blk_copy(lambda o, n: acc_sh.at[pl.ds(base + o, n)],
            lambda o, n: out_hbm.at[cid, pl.ds(base + o, n)])


@functools.partial(
    pl.kernel,
    out_type=jax.ShapeDtypeStruct((NC, NACC, W2), jnp.float32),
    mesh=_MESH,
    compiler_params=pltpu.CompilerParams(use_tc_tiling_on_sc=False),
    scratch_types=[
        pltpu.VMEM((NCHUNK2, B), jnp.int32),
        pltpu.VMEM((NCHUNK2, B), jnp.int32),
        pltpu.VMEM((B, W2), jnp.float32),
        pltpu.VMEM((B, W2), jnp.float32),
        pltpu.VMEM_SHARED((NACC, W2), jnp.float32),
        pltpu.SemaphoreType.DMA,
    ],
)
def _sc_agg2(table_hbm, sidx_hbm, didx_hbm, out_hbm,
             sidx_v, didx_v, gbuf, zbuf, acc_sh, sem):
  cid = lax.axis_index("c")
  sid = lax.axis_index("s")
  wid = sid * NC + cid

  _zero_fill(zbuf, W2)
  base = sid * RPT
  _blk_copy(lambda o, n: zbuf.at[pl.ds(0, n)],
            lambda o, n: acc_sh.at[pl.ds(base + o, n)])

  pltpu.sync_copy(sidx_hbm.at[wid], sidx_v)
  pltpu.sync_copy(didx_hbm.at[wid], didx_v)
  plsc.subcore_barrier()

  def chunk(j, _):
    pltpu.async_copy(table_hbm.at[sidx_v.at[j]], gbuf, sem).wait()
    pltpu.sync_copy(gbuf, acc_sh.at[didx_v.at[j]], add=True)
    return 0
  lax.fori_loop(0, NCHUNK2, chunk, 0)
  plsc.subcore_barrier()

  _blk_copy(lambda o, n: acc_sh.at[pl.ds(base + o, n)],
            lambda o, n: out_hbm.at[cid, pl.ds(base + o, n)])


def _tc_a_body(x_ref, wl_ref, wr_ref, b_ref, y_ref, z_ref):
  xb = x_ref[...]
  y = lax.dot_general(xb, wl_ref[...], (((1,), (1,)), ((), ())),
                      preferred_element_type=jnp.float32)
  y_ref[0] = y[:, :WH]
  y_ref[1] = jnp.concatenate(
      [y[:, WH:NFEAT],
       jnp.ones((BN, 16), jnp.float32),
       jnp.zeros((BN, WH - (NFEAT - WH) - 16), jnp.float32)], axis=1)
  z_ref[...] = lax.dot_general(xb, wr_ref[...], (((1,), (1,)), ((), ())),
                               preferred_element_type=jnp.float32) + b_ref[...]


def _tc_b_body(p1_ref, z1_ref, sc_ref, sh_ref, wl_ref, wr_ref, b2_ref,
               y2_ref, z2_ref):
  feat = jnp.concatenate([p1_ref[0], p1_ref[1][:, :NFEAT - WH]], axis=1)
  deg = jnp.maximum(p1_ref[1][:, NFEAT - WH:NFEAT - WH + 1], 1.0)
  deginv = 1.0 / deg
  h = feat * deginv + z1_ref[...]
  h = jax.nn.relu(h * sc_ref[...] + sh_ref[...])
  y2_ref[...] = lax.dot_general(h, wl_ref[...], (((1,), (1,)), ((), ())),
                                preferred_element_type=jnp.float32)
  z2 = lax.dot_general(h, wr_ref[...], (((1,), (1,)), ((), ())),
                       preferred_element_type=jnp.float32) + b2_ref[...]
  col = lax.broadcasted_iota(jnp.int32, (BN, W2), 1)
  z2_ref[...] = jnp.where(col == NCLASS, deginv, z2)


def _tc_c_body(p2_ref, z2_ref, o_ref):
  acc = p2_ref[0] + p2_ref[1]
  z2 = z2_ref[...]
  deginv = z2[:, NCLASS:NCLASS + 1]
  o = acc * deginv + z2
  col = lax.broadcasted_iota(jnp.int32, (BN, W2), 1)
  valid = col < NCLASS
  om = jnp.where(valid, o, -jnp.inf)
  m = jnp.max(om, axis=1, keepdims=True)
  s = jnp.sum(jnp.where(valid, jnp.exp(om - m), 0.0), axis=1, keepdims=True)
  out = o - m - jnp.log(s)
  o_ref[...] = out[:, :NCLASS]


def kernel(x, edge_index, Wl1, Wr1, b1, gamma, beta, rm, rv, Wl2, Wr2, b2):
  src = edge_index[0]
  dst = edge_index[1]
  pad = EPAD - E
  srcp = jnp.concatenate([src, jnp.zeros((pad,), jnp.int32)])
  dstp = jnp.concatenate([dst, jnp.full((pad,), N, jnp.int32)])
  sidx1 = srcp.reshape(NS, NCHUNK1, B)
  didx1 = dstp.reshape(NS, NCHUNK1, B)
  sidx2 = srcp.reshape(NW, NCHUNK2, B)
  didx2 = dstp.reshape(NW, NCHUNK2, B)

  bn_mul = gamma * lax.rsqrt(rv + EPS)
  bn_scale = bn_mul.reshape(1, NHID)
  bn_shift = (beta - rm * bn_mul).reshape(1, NHID)
  b1r = b1.reshape(1, NHID)
  Wl2p = jnp.zeros((W2, NHID), jnp.float32).at[:NCLASS].set(Wl2)
  Wr2p = jnp.zeros((W2, NHID), jnp.float32).at[:NCLASS].set(Wr2)
  b2p = jnp.zeros((1, W2), jnp.float32).at[0, :NCLASS].set(b2)

  full = lambda shape: pl.BlockSpec(shape, lambda i: (0,) * len(shape))

  y1, z1 = pl.pallas_call(
      _tc_a_body,
      grid=(GRID,),
      in_specs=[
          pl.BlockSpec((BN, NFEAT), lambda i: (i, 0)),
          full((NHID, NFEAT)), full((NHID, NFEAT)), full((1, NHID)),
      ],
      out_specs=[
          pl.BlockSpec((NC, BN, WH), lambda i: (0, i, 0)),
          pl.BlockSpec((BN, NHID), lambda i: (i, 0)),
      ],
      out_shape=[
          jax.ShapeDtypeStruct((NC, N, WH), jnp.float32),
          jax.ShapeDtypeStruct((N, NHID), jnp.float32),
      ],
  )(x, Wl1, Wr1, b1r)

  p1 = _sc_agg1(y1, sidx1, didx1)

  y2, z2 = pl.pallas_call(
      _tc_b_body,
      grid=(GRID,),
      in_specs=[
          pl.BlockSpec((NC, BN, WH), lambda i: (0, i, 0)),
          pl.BlockSpec((BN, NHID), lambda i: (i, 0)),
          full((1, NHID)), full((1, NHID)),
          full((W2, NHID)), full((W2, NHID)), full((1, W2)),
      ],
      out_specs=[
          pl.BlockSpec((BN, W2), lambda i: (i, 0)),
          pl.BlockSpec((BN, W2), lambda i: (i, 0)),
      ],
      out_shape=[
          jax.ShapeDtypeStruct((N, W2), jnp.float32),
          jax.ShapeDtypeStruct((N, W2), jnp.float32),
      ],
  )(p1, z1, bn_scale, bn_shift, Wl2p, Wr2p, b2p)

  p2 = _sc_agg2(y2, sidx2, didx2)

  out = pl.pallas_call(
      _tc_c_body,
      grid=(GRID,),
      in_specs=[
          pl.BlockSpec((NC, BN, W2), lambda i: (0, i, 0)),
          pl.BlockSpec((BN, W2), lambda i: (i, 0)),
      ],
      out_specs=pl.BlockSpec((BN, NCLASS), lambda i: (i, 0)),
      out_shape=jax.ShapeDtypeStruct((N, NCLASS), jnp.float32),
  )(p2, z2)
  return out


# speedup vs baseline: 6.3499x; 1.1241x over previous
import functools

import jax
import jax.numpy as jnp
from jax import lax
from jax.experimental import pallas as pl
from jax.experimental.pallas import tpu as pltpu
from jax.experimental.pallas import tpu_sc as plsc

N = 10000
E = 320000
NFEAT = 128
NHID = 128
NCLASS = 47
EPS = 1e-5

NC = 2
NS = 16
NW = NC * NS
B = 128
NCHUNK1 = 158
NCHUNK2 = 79
EPAD = NS * NCHUNK1 * B
WH = 80
W2 = 48
RPT = 632
NACC = NS * RPT

BN = 1000
GRID = N // BN


def _edge_loop(nchunk, table, sidx_v, didx_v, gbuf, acc_sh, sem):
  pltpu.async_copy(table.at[sidx_v.at[0]], gbuf.at[0], sem)
  def chunk(j, _):
    b = j % 2
    nxt = jnp.minimum(j + 1, nchunk - 1)
    pltpu.make_async_copy(table.at[sidx_v.at[j]], gbuf.at[b], sem).wait()
    pltpu.async_copy(table.at[sidx_v.at[nxt]], gbuf.at[1 - b], sem)
    pltpu.sync_copy(gbuf.at[b], acc_sh.at[didx_v.at[j]], add=True)
    return 0
  lax.fori_loop(0, nchunk, chunk, 0)
  pltpu.make_async_copy(table.at[sidx_v.at[0]], gbuf.at[0], sem).wait()


def _zero_fill(zbuf, width):
  zero16 = jnp.zeros((16,), jnp.float32)
  def zrow(i, _):
    for j in range(width // 16):
      zbuf[i, pl.ds(j * 16, 16)] = zero16
    return 0
  lax.fori_loop(0, B, zrow, 0)


def _blk_copy(src, dst):
  for k in range(RPT // B):
    pltpu.sync_copy(src(k * B, B), dst(k * B, B))
  rem = RPT % B
  if rem:
    pltpu.sync_copy(src((RPT // B) * B, rem), dst((RPT // B) * B, rem))


_MESH = plsc.VectorSubcoreMesh(core_axis_name="c", subcore_axis_name="s")


@functools.partial(
    pl.kernel,
    out_type=jax.ShapeDtypeStruct((NC, NACC, WH), jnp.float32),
    mesh=_MESH,
    compiler_params=pltpu.CompilerParams(use_tc_tiling_on_sc=False),
    scratch_types=[
        pltpu.VMEM((NCHUNK1, B), jnp.int32),
        pltpu.VMEM((NCHUNK1, B), jnp.int32),
        pltpu.VMEM((2, B, WH), jnp.float32),
        pltpu.VMEM((B, WH), jnp.float32),
        pltpu.VMEM_SHARED((NACC, WH), jnp.float32),
        pltpu.SemaphoreType.DMA,
    ],
)
def _sc_agg1(table_hbm, sidx_hbm, didx_hbm, out_hbm,
             sidx_v, didx_v, gbuf, zbuf, acc_sh, sem):
  cid = lax.axis_index("c")
  sid = lax.axis_index("s")

  _zero_fill(zbuf, WH)
  base = sid * RPT
  _blk_copy(lambda o, n: zbuf.at[pl.ds(0, n)],
            lambda o, n: acc_sh.at[pl.ds(base + o, n)])

  pltpu.sync_copy(sidx_hbm.at[sid], sidx_v)
  pltpu.sync_copy(didx_hbm.at[sid], didx_v)
  plsc.subcore_barrier()

  _edge_loop(NCHUNK1, table_hbm.at[cid], sidx_v, didx_v, gbuf, acc_sh, sem)
  plsc.subcore_barrier()

  _blk_copy(lambda o, n: acc_sh.at[pl.ds(base + o, n)],
            lambda o, n: out_hbm.at[cid, pl.ds(base + o, n)])


@functools.partial(
    pl.kernel,
    out_type=jax.ShapeDtypeStruct((NC, NACC, W2), jnp.float32),
    mesh=_MESH,
    compiler_params=pltpu.CompilerParams(use_tc_tiling_on_sc=False),
    scratch_types=[
        pltpu.VMEM((NCHUNK2, B), jnp.int32),
        pltpu.VMEM((NCHUNK2, B), jnp.int32),
        pltpu.VMEM((2, B, W2), jnp.float32),
        pltpu.VMEM((B, W2), jnp.float32),
        pltpu.VMEM_SHARED((NACC, W2), jnp.float32),
        pltpu.SemaphoreType.DMA,
    ],
)
def _sc_agg2(table_hbm, sidx_hbm, didx_hbm, out_hbm,
             sidx_v, didx_v, gbuf, zbuf, acc_sh, sem):
  cid = lax.axis_index("c")
  sid = lax.axis_index("s")
  wid = sid * NC + cid

  _zero_fill(zbuf, W2)
  base = sid * RPT
  _blk_copy(lambda o, n: zbuf.at[pl.ds(0, n)],
            lambda o, n: acc_sh.at[pl.ds(base + o, n)])

  pltpu.sync_copy(sidx_hbm.at[wid], sidx_v)
  pltpu.sync_copy(didx_hbm.at[wid], didx_v)
  plsc.subcore_barrier()

  _edge_loop(NCHUNK2, table_hbm, sidx_v, didx_v, gbuf, acc_sh, sem)
  plsc.subcore_barrier()

  _blk_copy(lambda o, n: acc_sh.at[pl.ds(base + o, n)],
            lambda o, n: out_hbm.at[cid, pl.ds(base + o, n)])


def _tc_a_body(x_ref, wl_ref, wr_ref, b_ref, y_ref, z_ref):
  xb = x_ref[...]
  y = lax.dot_general(xb, wl_ref[...], (((1,), (1,)), ((), ())),
                      preferred_element_type=jnp.float32)
  y_ref[0] = y[:, :WH]
  y_ref[1] = jnp.concatenate(
      [y[:, WH:NFEAT],
       jnp.ones((BN, 16), jnp.float32),
       jnp.zeros((BN, WH - (NFEAT - WH) - 16), jnp.float32)], axis=1)
  z_ref[...] = lax.dot_general(xb, wr_ref[...], (((1,), (1,)), ((), ())),
                               preferred_element_type=jnp.float32) + b_ref[...]


def _tc_b_body(p1_ref, z1_ref, sc_ref, sh_ref, wl_ref, wr_ref, b2_ref,
               y2_ref, z2_ref):
  feat = jnp.concatenate([p1_ref[0], p1_ref[1][:, :NFEAT - WH]], axis=1)
  deg = jnp.maximum(p1_ref[1][:, NFEAT - WH:NFEAT - WH + 1], 1.0)
  deginv = 1.0 / deg
  h = feat * deginv + z1_ref[...]
  h = jax.nn.relu(h * sc_ref[...] + sh_ref[...])
  y2_ref[...] = lax.dot_general(h, wl_ref[...], (((1,), (1,)), ((), ())),
                                preferred_element_type=jnp.float32)
  z2 = lax.dot_general(h, wr_ref[...], (((1,), (1,)), ((), ())),
                       preferred_element_type=jnp.float32) + b2_ref[...]
  col = lax.broadcasted_iota(jnp.int32, (BN, W2), 1)
  z2_ref[...] = jnp.where(col == NCLASS, deginv, z2)


def _tc_c_body(p2_ref, z2_ref, o_ref):
  acc = p2_ref[0] + p2_ref[1]
  z2 = z2_ref[...]
  deginv = z2[:, NCLASS:NCLASS + 1]
  o = acc * deginv + z2
  col = lax.broadcasted_iota(jnp.int32, (BN, W2), 1)
  valid = col < NCLASS
  om = jnp.where(valid, o, -jnp.inf)
  m = jnp.max(om, axis=1, keepdims=True)
  s = jnp.sum(jnp.where(valid, jnp.exp(om - m), 0.0), axis=1, keepdims=True)
  out = o - m - jnp.log(s)
  o_ref[...] = out[:, :NCLASS]


def kernel(x, edge_index, Wl1, Wr1, b1, gamma, beta, rm, rv, Wl2, Wr2, b2):
  src = edge_index[0]
  dst = edge_index[1]
  pad = EPAD - E
  srcp = jnp.concatenate([src, jnp.zeros((pad,), jnp.int32)])
  dstp = jnp.concatenate([dst, jnp.full((pad,), N, jnp.int32)])
  sidx1 = srcp.reshape(NS, NCHUNK1, B)
  didx1 = dstp.reshape(NS, NCHUNK1, B)
  sidx2 = srcp.reshape(NW, NCHUNK2, B)
  didx2 = dstp.reshape(NW, NCHUNK2, B)

  bn_mul = gamma * lax.rsqrt(rv + EPS)
  bn_scale = bn_mul.reshape(1, NHID)
  bn_shift = (beta - rm * bn_mul).reshape(1, NHID)
  b1r = b1.reshape(1, NHID)
  Wl2p = jnp.zeros((W2, NHID), jnp.float32).at[:NCLASS].set(Wl2)
  Wr2p = jnp.zeros((W2, NHID), jnp.float32).at[:NCLASS].set(Wr2)
  b2p = jnp.zeros((1, W2), jnp.float32).at[0, :NCLASS].set(b2)

  full = lambda shape: pl.BlockSpec(shape, lambda i: (0,) * len(shape))

  y1, z1 = pl.pallas_call(
      _tc_a_body,
      grid=(GRID,),
      in_specs=[
          pl.BlockSpec((BN, NFEAT), lambda i: (i, 0)),
          full((NHID, NFEAT)), full((NHID, NFEAT)), full((1, NHID)),
      ],
      out_specs=[
          pl.BlockSpec((NC, BN, WH), lambda i: (0, i, 0)),
          pl.BlockSpec((BN, NHID), lambda i: (i, 0)),
      ],
      out_shape=[
          jax.ShapeDtypeStruct((NC, N, WH), jnp.float32),
          jax.ShapeDtypeStruct((N, NHID), jnp.float32),
      ],
  )(x, Wl1, Wr1, b1r)

  p1 = _sc_agg1(y1, sidx1, didx1)

  y2, z2 = pl.pallas_call(
      _tc_b_body,
      grid=(GRID,),
      in_specs=[
          pl.BlockSpec((NC, BN, WH), lambda i: (0, i, 0)),
          pl.BlockSpec((BN, NHID), lambda i: (i, 0)),
          full((1, NHID)), full((1, NHID)),
          full((W2, NHID)), full((W2, NHID)), full((1, W2)),
      ],
      out_specs=[
          pl.BlockSpec((BN, W2), lambda i: (i, 0)),
          pl.BlockSpec((BN, W2), lambda i: (i, 0)),
      ],
      out_shape=[
          jax.ShapeDtypeStruct((N, W2), jnp.float32),
          jax.ShapeDtypeStruct((N, W2), jnp.float32),
      ],
  )(p1, z1, bn_scale, bn_shift, Wl2p, Wr2p, b2p)

  p2 = _sc_agg2(y2, sidx2, didx2)

  out = pl.pallas_call(
      _tc_c_body,
      grid=(GRID,),
      in_specs=[
          pl.BlockSpec((NC, BN, W2), lambda i: (0, i, 0)),
          pl.BlockSpec((BN, W2), lambda i: (i, 0)),
      ],
      out_specs=pl.BlockSpec((BN, NCLASS), lambda i: (i, 0)),
      out_shape=jax.ShapeDtypeStruct((N, NCLASS), jnp.float32),
  )(p2, z2)
  return out


# speedup vs baseline: 7.6420x; 1.2035x over previous
import functools

import jax
import jax.numpy as jnp
from jax import lax
from jax.experimental import pallas as pl
from jax.experimental.pallas import tpu as pltpu
from jax.experimental.pallas import tpu_sc as plsc

N = 10000
E = 320000
NFEAT = 128
NHID = 128
NCLASS = 47
EPS = 1e-5

NC = 2
NS = 16
NW = NC * NS
B = 128
G = 1
GB = G * B
NG1 = 158
NG2 = 79
EPAD = NS * NG1 * GB
WH = 80
W2 = 48
RPT = 632
NACC = NS * RPT

BN = 1000
GRID = N // BN


NBUF = 6
GA = 4
SO = 2
IR = 7


def _edge_loop(ngrp, table, idxh, idxr, gbuf, acc_sh, sem, sem_i, sem_s):
  for b in range(GA + 1):
    pltpu.async_copy(idxh.at[b], idxr.at[b], sem_i)
  for b in range(GA):
    pltpu.make_async_copy(idxh.at[b], idxr.at[b], sem_i).wait()
    pltpu.async_copy(table.at[idxr.at[b, 0]], gbuf.at[b], sem)

  def grp(j, _):
    @pl.when(j >= SO)
    def _wait_scatter():
      pltpu.make_async_copy(gbuf.at[0], acc_sh.at[idxr.at[0, 1]],
                            sem_s).wait()
    nip = jnp.minimum(j + GA + 1, ngrp - 1)
    pltpu.async_copy(idxh.at[nip], idxr.at[(j + GA + 1) % IR], sem_i)
    c = jnp.minimum(j + GA, ngrp - 1)
    pltpu.make_async_copy(idxh.at[0], idxr.at[0], sem_i).wait()
    pltpu.async_copy(table.at[idxr.at[c % IR, 0]], gbuf.at[(j + GA) % NBUF],
                     sem)
    b = j % NBUF
    pltpu.make_async_copy(table.at[idxr.at[0, 0]], gbuf.at[b], sem).wait()
    pltpu.async_copy(gbuf.at[b], acc_sh.at[idxr.at[j % IR, 1]], sem_s,
                     add=True)
    return 0
  lax.fori_loop(0, ngrp, grp, 0)

  pltpu.make_async_copy(idxh.at[0], idxr.at[0], sem_i).wait()
  for _ in range(GA):
    pltpu.make_async_copy(table.at[idxr.at[0, 0]], gbuf.at[0], sem).wait()
  for _ in range(SO):
    pltpu.make_async_copy(gbuf.at[0], acc_sh.at[idxr.at[0, 1]], sem_s).wait()


def _zero_acc(gbuf, acc_sh, base, width):
  zflat = gbuf.at[0]
  zero16 = jnp.zeros((16,), jnp.float32)
  def zrow(i, _):
    for j in range(width // 16):
      zflat[i, pl.ds(j * 16, 16)] = zero16
    return 0
  lax.fori_loop(0, GB, zrow, 0)
  for k in range(RPT // GB):
    pltpu.sync_copy(zflat, acc_sh.at[pl.ds(base + k * GB, GB)])
  rem = RPT % GB
  if rem:
    pltpu.sync_copy(zflat.at[pl.ds(0, rem)],
                    acc_sh.at[pl.ds(base + (RPT // GB) * GB, rem)])


def _copy_out(acc_sh, out_hbm, cid, base):
  for k in range(RPT // B):
    pltpu.sync_copy(acc_sh.at[pl.ds(base + k * B, B)],
                    out_hbm.at[cid, pl.ds(base + k * B, B)])
  rem = RPT % B
  if rem:
    pltpu.sync_copy(acc_sh.at[pl.ds(base + (RPT // B) * B, rem)],
                    out_hbm.at[cid, pl.ds(base + (RPT // B) * B, rem)])


_MESH = plsc.VectorSubcoreMesh(core_axis_name="c", subcore_axis_name="s")


@functools.partial(
    pl.kernel,
    out_type=jax.ShapeDtypeStruct((NC, NACC, WH), jnp.float32),
    mesh=_MESH,
    compiler_params=pltpu.CompilerParams(use_tc_tiling_on_sc=False),
    scratch_types=[
        pltpu.VMEM((IR, 2, GB), jnp.int32),
        pltpu.VMEM((NBUF, GB, WH), jnp.float32),
        pltpu.VMEM_SHARED((NACC, WH), jnp.float32),
        pltpu.SemaphoreType.DMA,
        pltpu.SemaphoreType.DMA,
        pltpu.SemaphoreType.DMA,
    ],
)
def _sc_agg1(table_hbm, idx_hbm, out_hbm,
             idxr, gbuf, acc_sh, sem, sem_i, sem_s):
  cid = lax.axis_index("c")
  sid = lax.axis_index("s")

  base = sid * RPT
  _zero_acc(gbuf, acc_sh, base, WH)
  plsc.subcore_barrier()

  _edge_loop(NG1, table_hbm.at[cid], idx_hbm.at[sid], idxr, gbuf, acc_sh,
             sem, sem_i, sem_s)
  plsc.subcore_barrier()

  _copy_out(acc_sh, out_hbm, cid, base)


@functools.partial(
    pl.kernel,
    out_type=jax.ShapeDtypeStruct((NC, NACC, W2), jnp.float32),
    mesh=_MESH,
    compiler_params=pltpu.CompilerParams(use_tc_tiling_on_sc=False),
    scratch_types=[
        pltpu.VMEM((IR, 2, GB), jnp.int32),
        pltpu.VMEM((NBUF, GB, W2), jnp.float32),
        pltpu.VMEM_SHARED((NACC, W2), jnp.float32),
        pltpu.SemaphoreType.DMA,
        pltpu.SemaphoreType.DMA,
        pltpu.SemaphoreType.DMA,
    ],
)
def _sc_agg2(table_hbm, idx_hbm, out_hbm,
             idxr, gbuf, acc_sh, sem, sem_i, sem_s):
  cid = lax.axis_index("c")
  sid = lax.axis_index("s")
  wid = sid * NC + cid

  base = sid * RPT
  _zero_acc(gbuf, acc_sh, base, W2)
  plsc.subcore_barrier()

  _edge_loop(NG2, table_hbm, idx_hbm.at[wid], idxr, gbuf, acc_sh,
             sem, sem_i, sem_s)
  plsc.subcore_barrier()

  _copy_out(acc_sh, out_hbm, cid, base)


def _tc_a_body(x_ref, wl_ref, wr_ref, b_ref, y_ref, z_ref):
  xb = x_ref[...]
  y = lax.dot_general(xb, wl_ref[...], (((1,), (1,)), ((), ())),
                      preferred_element_type=jnp.float32)
  y_ref[0] = y[:, :WH]
  y_ref[1] = jnp.concatenate(
      [y[:, WH:NFEAT],
       jnp.ones((BN, 16), jnp.float32),
       jnp.zeros((BN, WH - (NFEAT - WH) - 16), jnp.float32)], axis=1)
  z_ref[...] = lax.dot_general(xb, wr_ref[...], (((1,), (1,)), ((), ())),
                               preferred_element_type=jnp.float32) + b_ref[...]


def _tc_b_body(p1_ref, z1_ref, sc_ref, sh_ref, wl_ref, wr_ref, b2_ref,
               y2_ref, z2_ref):
  feat = jnp.concatenate([p1_ref[0], p1_ref[1][:, :NFEAT - WH]], axis=1)
  deg = jnp.maximum(p1_ref[1][:, NFEAT - WH:NFEAT - WH + 1], 1.0)
  deginv = 1.0 / deg
  h = feat * deginv + z1_ref[...]
  h = jax.nn.relu(h * sc_ref[...] + sh_ref[...])
  y2_ref[...] = lax.dot_general(h, wl_ref[...], (((1,), (1,)), ((), ())),
                                preferred_element_type=jnp.float32)
  z2 = lax.dot_general(h, wr_ref[...], (((1,), (1,)), ((), ())),
                       preferred_element_type=jnp.float32) + b2_ref[...]
  col = lax.broadcasted_iota(jnp.int32, (BN, W2), 1)
  z2_ref[...] = jnp.where(col == NCLASS, deginv, z2)


def _tc_c_body(p2_ref, z2_ref, o_ref):
  acc = p2_ref[0] + p2_ref[1]
  z2 = z2_ref[...]
  deginv = z2[:, NCLASS:NCLASS + 1]
  o = acc * deginv + z2
  col = lax.broadcasted_iota(jnp.int32, (BN, W2), 1)
  valid = col < NCLASS
  om = jnp.where(valid, o, -jnp.inf)
  m = jnp.max(om, axis=1, keepdims=True)
  s = jnp.sum(jnp.where(valid, jnp.exp(om - m), 0.0), axis=1, keepdims=True)
  out = o - m - jnp.log(s)
  o_ref[...] = out[:, :NCLASS]


def kernel(x, edge_index, Wl1, Wr1, b1, gamma, beta, rm, rv, Wl2, Wr2, b2):
  src = edge_index[0]
  dst = edge_index[1]
  pad = EPAD - E
  srcp = jnp.concatenate([src, jnp.zeros((pad,), jnp.int32)])
  dstp = jnp.concatenate([dst, jnp.full((pad,), N, jnp.int32)])
  idx1 = jnp.stack([srcp.reshape(NS, NG1, GB),
                    dstp.reshape(NS, NG1, GB)], axis=2)
  idx2 = jnp.stack([srcp.reshape(NW, NG2, GB),
                    dstp.reshape(NW, NG2, GB)], axis=2)

  bn_mul = gamma * lax.rsqrt(rv + EPS)
  bn_scale = bn_mul.reshape(1, NHID)
  bn_shift = (beta - rm * bn_mul).reshape(1, NHID)
  b1r = b1.reshape(1, NHID)
  Wl2p = jnp.zeros((W2, NHID), jnp.float32).at[:NCLASS].set(Wl2)
  Wr2p = jnp.zeros((W2, NHID), jnp.float32).at[:NCLASS].set(Wr2)
  b2p = jnp.zeros((1, W2), jnp.float32).at[0, :NCLASS].set(b2)

  full = lambda shape: pl.BlockSpec(shape, lambda i: (0,) * len(shape))

  y1, z1 = pl.pallas_call(
      _tc_a_body,
      grid=(GRID,),
      in_specs=[
          pl.BlockSpec((BN, NFEAT), lambda i: (i, 0)),
          full((NHID, NFEAT)), full((NHID, NFEAT)), full((1, NHID)),
      ],
      out_specs=[
          pl.BlockSpec((NC, BN, WH), lambda i: (0, i, 0)),
          pl.BlockSpec((BN, NHID), lambda i: (i, 0)),
      ],
      out_shape=[
          jax.ShapeDtypeStruct((NC, N, WH), jnp.float32),
          jax.ShapeDtypeStruct((N, NHID), jnp.float32),
      ],
  )(x, Wl1, Wr1, b1r)

  p1 = _sc_agg1(y1, idx1)

  y2, z2 = pl.pallas_call(
      _tc_b_body,
      grid=(GRID,),
      in_specs=[
          pl.BlockSpec((NC, BN, WH), lambda i: (0, i, 0)),
          pl.BlockSpec((BN, NHID), lambda i: (i, 0)),
          full((1, NHID)), full((1, NHID)),
          full((W2, NHID)), full((W2, NHID)), full((1, W2)),
      ],
      out_specs=[
          pl.BlockSpec((BN, W2), lambda i: (i, 0)),
          pl.BlockSpec((BN, W2), lambda i: (i, 0)),
      ],
      out_shape=[
          jax.ShapeDtypeStruct((N, W2), jnp.float32),
          jax.ShapeDtypeStruct((N, W2), jnp.float32),
      ],
  )(p1, z1, bn_scale, bn_shift, Wl2p, Wr2p, b2p)

  p2 = _sc_agg2(y2, idx2)

  out = pl.pallas_call(
      _tc_c_body,
      grid=(GRID,),
      in_specs=[
          pl.BlockSpec((NC, BN, W2), lambda i: (0, i, 0)),
          pl.BlockSpec((BN, W2), lambda i: (i, 0)),
      ],
      out_specs=pl.BlockSpec((BN, NCLASS), lambda i: (i, 0)),
      out_shape=jax.ShapeDtypeStruct((N, NCLASS), jnp.float32),
  )(p2, z2)
  return out


# speedup vs baseline: 10.3637x; 1.3562x over previous
import functools

import jax
import jax.numpy as jnp
from jax import lax
from jax.experimental import pallas as pl
from jax.experimental.pallas import tpu as pltpu
from jax.experimental.pallas import tpu_sc as plsc

N = 10000
E = 320000
NFEAT = 128
NHID = 128
NCLASS = 47
EPS = 1e-5

NC = 2
NS = 16
NW = NC * NS
B = 128
G = 1
GB = G * B
NG1 = 158
NG2 = 79
EPAD = NS * NG1 * GB
WH = 80
WA = 96
WB = 64
W2 = 48
RPT = 632
NACC = NS * RPT

BN = 1000
GRID = N // BN


NBUF = 6
GA = 4
SO = 2
IR = 7


def _edge_loop(ngrp, table, idxh, idxr, gbuf, acc_sh, sem, sem_i, sem_s):
  for b in range(GA + 1):
    pltpu.async_copy(idxh.at[b], idxr.at[b], sem_i)
  for b in range(GA):
    pltpu.make_async_copy(idxh.at[b], idxr.at[b], sem_i).wait()
    pltpu.async_copy(table.at[idxr.at[b, 0]], gbuf.at[b], sem)

  def grp(j, _):
    @pl.when(j >= SO)
    def _wait_scatter():
      pltpu.make_async_copy(gbuf.at[0], acc_sh.at[idxr.at[0, 1]],
                            sem_s).wait()
    nip = jnp.minimum(j + GA + 1, ngrp - 1)
    pltpu.async_copy(idxh.at[nip], idxr.at[(j + GA + 1) % IR], sem_i)
    c = jnp.minimum(j + GA, ngrp - 1)
    pltpu.make_async_copy(idxh.at[0], idxr.at[0], sem_i).wait()
    pltpu.async_copy(table.at[idxr.at[c % IR, 0]], gbuf.at[(j + GA) % NBUF],
                     sem)
    b = j % NBUF
    pltpu.make_async_copy(table.at[idxr.at[0, 0]], gbuf.at[b], sem).wait()
    pltpu.async_copy(gbuf.at[b], acc_sh.at[idxr.at[j % IR, 1]], sem_s,
                     add=True)
    return 0
  lax.fori_loop(0, ngrp, grp, 0)

  pltpu.make_async_copy(idxh.at[0], idxr.at[0], sem_i).wait()
  for _ in range(GA):
    pltpu.make_async_copy(table.at[idxr.at[0, 0]], gbuf.at[0], sem).wait()
  for _ in range(SO):
    pltpu.make_async_copy(gbuf.at[0], acc_sh.at[idxr.at[0, 1]], sem_s).wait()


def _zero_acc(gbuf, acc_sh, base, width):
  zflat = gbuf.at[0]
  zero16 = jnp.zeros((16,), jnp.float32)
  def zrow(i, _):
    for j in range(width // 16):
      zflat[i, pl.ds(j * 16, 16)] = zero16
    return 0
  lax.fori_loop(0, GB, zrow, 0)
  for k in range(RPT // GB):
    pltpu.sync_copy(zflat, acc_sh.at[pl.ds(base + k * GB, GB)])
  rem = RPT % GB
  if rem:
    pltpu.sync_copy(zflat.at[pl.ds(0, rem)],
                    acc_sh.at[pl.ds(base + (RPT // GB) * GB, rem)])


def _zero_acc_bf(gbuf, acc_sh, base, width):
  zflat = gbuf.at[0]
  zero32 = jnp.zeros((32,), jnp.bfloat16)
  def zrow(i, _):
    for j in range(width // 32):
      zflat[i, pl.ds(j * 32, 32)] = zero32
    return 0
  lax.fori_loop(0, GB, zrow, 0)
  for k in range(RPT // GB):
    pltpu.sync_copy(zflat, acc_sh.at[pl.ds(base + k * GB, GB)])
  rem = RPT % GB
  if rem:
    pltpu.sync_copy(zflat.at[pl.ds(0, rem)],
                    acc_sh.at[pl.ds(base + (RPT // GB) * GB, rem)])


def _copy_out(acc_sh, out_hbm, cid, base):
  for k in range(RPT // B):
    pltpu.sync_copy(acc_sh.at[pl.ds(base + k * B, B)],
                    out_hbm.at[cid, pl.ds(base + k * B, B)])
  rem = RPT % B
  if rem:
    pltpu.sync_copy(acc_sh.at[pl.ds(base + (RPT // B) * B, rem)],
                    out_hbm.at[cid, pl.ds(base + (RPT // B) * B, rem)])


def _copy_out2(acc_sh, out_hbm, base):
  for k in range(RPT // B):
    pltpu.sync_copy(acc_sh.at[pl.ds(base + k * B, B)],
                    out_hbm.at[pl.ds(base + k * B, B)])
  rem = RPT % B
  if rem:
    pltpu.sync_copy(acc_sh.at[pl.ds(base + (RPT // B) * B, rem)],
                    out_hbm.at[pl.ds(base + (RPT // B) * B, rem)])


_MESH = plsc.VectorSubcoreMesh(core_axis_name="c", subcore_axis_name="s")


@functools.partial(
    pl.kernel,
    out_type=(jax.ShapeDtypeStruct((NACC, WA), jnp.bfloat16),
              jax.ShapeDtypeStruct((NACC, WB), jnp.bfloat16)),
    mesh=_MESH,
    compiler_params=pltpu.CompilerParams(use_tc_tiling_on_sc=False),
    scratch_types=[
        pltpu.VMEM((IR, 2, GB), jnp.int32),
        pltpu.VMEM((NBUF, GB, WA), jnp.bfloat16),
        pltpu.VMEM((NBUF, GB, WB), jnp.bfloat16),
        pltpu.VMEM_SHARED((NACC, WA), jnp.bfloat16),
        pltpu.VMEM_SHARED((NACC, WB), jnp.bfloat16),
        pltpu.SemaphoreType.DMA,
        pltpu.SemaphoreType.DMA,
        pltpu.SemaphoreType.DMA,
    ],
)
def _sc_agg1(t0_hbm, t1_hbm, idx_hbm, o0_hbm, o1_hbm,
             idxr, gbuf0, gbuf1, acc0, acc1, sem, sem_i, sem_s):
  cid = lax.axis_index("c")
  sid = lax.axis_index("s")

  base = sid * RPT
  @pl.when(cid == 0)
  def _z0():
    _zero_acc_bf(gbuf0, acc0, base, WA)
  @pl.when(cid == 1)
  def _z1():
    _zero_acc_bf(gbuf1, acc1, base, WB)
  plsc.subcore_barrier()

  @pl.when(cid == 0)
  def _e0():
    _edge_loop(NG1, t0_hbm, idx_hbm.at[sid], idxr, gbuf0, acc0,
               sem, sem_i, sem_s)
  @pl.when(cid == 1)
  def _e1():
    _edge_loop(NG1, t1_hbm, idx_hbm.at[sid], idxr, gbuf1, acc1,
               sem, sem_i, sem_s)
  plsc.subcore_barrier()

  @pl.when(cid == 0)
  def _c0():
    _copy_out2(acc0, o0_hbm, base)
  @pl.when(cid == 1)
  def _c1():
    _copy_out2(acc1, o1_hbm, base)


@functools.partial(
    pl.kernel,
    out_type=jax.ShapeDtypeStruct((NC, NACC, W2), jnp.float32),
    mesh=_MESH,
    compiler_params=pltpu.CompilerParams(use_tc_tiling_on_sc=False),
    scratch_types=[
        pltpu.VMEM((IR, 2, GB), jnp.int32),
        pltpu.VMEM((NBUF, GB, W2), jnp.float32),
        pltpu.VMEM_SHARED((NACC, W2), jnp.float32),
        pltpu.SemaphoreType.DMA,
        pltpu.SemaphoreType.DMA,
        pltpu.SemaphoreType.DMA,
    ],
)
def _sc_agg2(table_hbm, idx_hbm, out_hbm,
             idxr, gbuf, acc_sh, sem, sem_i, sem_s):
  cid = lax.axis_index("c")
  sid = lax.axis_index("s")
  wid = sid * NC + cid

  base = sid * RPT
  _zero_acc(gbuf, acc_sh, base, W2)
  plsc.subcore_barrier()

  _edge_loop(NG2, table_hbm, idx_hbm.at[wid], idxr, gbuf, acc_sh,
             sem, sem_i, sem_s)
  plsc.subcore_barrier()

  _copy_out(acc_sh, out_hbm, cid, base)


def _tc_a_body(x_ref, wl_ref, wr_ref, b_ref, t0_ref, t1_ref, z_ref):
  xb = x_ref[...]
  y = lax.dot_general(xb, wl_ref[...], (((1,), (1,)), ((), ())),
                      preferred_element_type=jnp.float32)
  t0_ref[...] = y[:, :WA].astype(jnp.bfloat16)
  t1_ref[...] = jnp.concatenate(
      [y[:, WA:NFEAT],
       jnp.ones((BN, 16), jnp.float32),
       jnp.zeros((BN, WB - (NFEAT - WA) - 16), jnp.float32)],
      axis=1).astype(jnp.bfloat16)
  z_ref[...] = lax.dot_general(xb, wr_ref[...], (((1,), (1,)), ((), ())),
                               preferred_element_type=jnp.float32) + b_ref[...]


def _tc_b_body(p0_ref, p1_ref, z1_ref, sc_ref, sh_ref, wl_ref, wr_ref,
               b2_ref, y2_ref, z2_ref):
  p0 = p0_ref[...].astype(jnp.float32)
  p1 = p1_ref[...].astype(jnp.float32)
  feat = jnp.concatenate([p0, p1[:, :NFEAT - WA]], axis=1)
  deg = jnp.maximum(p1[:, NFEAT - WA:NFEAT - WA + 1], 1.0)
  deginv = 1.0 / deg
  h = feat * deginv + z1_ref[...]
  h = jax.nn.relu(h * sc_ref[...] + sh_ref[...])
  y2_ref[...] = lax.dot_general(h, wl_ref[...], (((1,), (1,)), ((), ())),
                                preferred_element_type=jnp.float32)
  z2 = lax.dot_general(h, wr_ref[...], (((1,), (1,)), ((), ())),
                       preferred_element_type=jnp.float32) + b2_ref[...]
  col = lax.broadcasted_iota(jnp.int32, (BN, W2), 1)
  z2_ref[...] = jnp.where(col == NCLASS, deginv, z2)


def _tc_c_body(p2_ref, z2_ref, o_ref):
  acc = p2_ref[0] + p2_ref[1]
  z2 = z2_ref[...]
  deginv = z2[:, NCLASS:NCLASS + 1]
  o = acc * deginv + z2
  col = lax.broadcasted_iota(jnp.int32, (BN, W2), 1)
  valid = col < NCLASS
  om = jnp.where(valid, o, -jnp.inf)
  m = jnp.max(om, axis=1, keepdims=True)
  s = jnp.sum(jnp.where(valid, jnp.exp(om - m), 0.0), axis=1, keepdims=True)
  out = o - m - jnp.log(s)
  o_ref[...] = out[:, :NCLASS]


def kernel(x, edge_index, Wl1, Wr1, b1, gamma, beta, rm, rv, Wl2, Wr2, b2):
  src = edge_index[0]
  dst = edge_index[1]
  pad = EPAD - E
  srcp = jnp.concatenate([src, jnp.zeros((pad,), jnp.int32)])
  dstp = jnp.concatenate([dst, jnp.full((pad,), N, jnp.int32)])
  idx1 = jnp.stack([srcp.reshape(NS, NG1, GB),
                    dstp.reshape(NS, NG1, GB)], axis=2)
  idx2 = jnp.stack([srcp.reshape(NW, NG2, GB),
                    dstp.reshape(NW, NG2, GB)], axis=2)

  bn_mul = gamma * lax.rsqrt(rv + EPS)
  bn_scale = bn_mul.reshape(1, NHID)
  bn_shift = (beta - rm * bn_mul).reshape(1, NHID)
  b1r = b1.reshape(1, NHID)
  Wl2p = jnp.zeros((W2, NHID), jnp.float32).at[:NCLASS].set(Wl2)
  Wr2p = jnp.zeros((W2, NHID), jnp.float32).at[:NCLASS].set(Wr2)
  b2p = jnp.zeros((1, W2), jnp.float32).at[0, :NCLASS].set(b2)

  full = lambda shape: pl.BlockSpec(shape, lambda i: (0,) * len(shape))

  t0, t1, z1 = pl.pallas_call(
      _tc_a_body,
      grid=(GRID,),
      in_specs=[
          pl.BlockSpec((BN, NFEAT), lambda i: (i, 0)),
          full((NHID, NFEAT)), full((NHID, NFEAT)), full((1, NHID)),
      ],
      out_specs=[
          pl.BlockSpec((BN, WA), lambda i: (i, 0)),
          pl.BlockSpec((BN, WB), lambda i: (i, 0)),
          pl.BlockSpec((BN, NHID), lambda i: (i, 0)),
      ],
      out_shape=[
          jax.ShapeDtypeStruct((N, WA), jnp.bfloat16),
          jax.ShapeDtypeStruct((N, WB), jnp.bfloat16),
          jax.ShapeDtypeStruct((N, NHID), jnp.float32),
      ],
  )(x, Wl1, Wr1, b1r)

  p0, p1 = _sc_agg1(t0, t1, idx1)

  y2, z2 = pl.pallas_call(
      _tc_b_body,
      grid=(GRID,),
      in_specs=[
          pl.BlockSpec((BN, WA), lambda i: (i, 0)),
          pl.BlockSpec((BN, WB), lambda i: (i, 0)),
          pl.BlockSpec((BN, NHID), lambda i: (i, 0)),
          full((1, NHID)), full((1, NHID)),
          full((W2, NHID)), full((W2, NHID)), full((1, W2)),
      ],
      out_specs=[
          pl.BlockSpec((BN, W2), lambda i: (i, 0)),
          pl.BlockSpec((BN, W2), lambda i: (i, 0)),
      ],
      out_shape=[
          jax.ShapeDtypeStruct((N, W2), jnp.float32),
          jax.ShapeDtypeStruct((N, W2), jnp.float32),
      ],
  )(p0, p1, z1, bn_scale, bn_shift, Wl2p, Wr2p, b2p)

  p2 = _sc_agg2(y2, idx2)

  out = pl.pallas_call(
      _tc_c_body,
      grid=(GRID,),
      in_specs=[
          pl.BlockSpec((NC, BN, W2), lambda i: (0, i, 0)),
          pl.BlockSpec((BN, W2), lambda i: (i, 0)),
      ],
      out_specs=pl.BlockSpec((BN, NCLASS), lambda i: (i, 0)),
      out_shape=jax.ShapeDtypeStruct((N, NCLASS), jnp.float32),
  )(p2, z2)
  return out


# speedup vs baseline: 11.8746x; 1.1458x over previous
import functools

import jax
import jax.numpy as jnp
from jax import lax
from jax.experimental import pallas as pl
from jax.experimental.pallas import tpu as pltpu
from jax.experimental.pallas import tpu_sc as plsc

N = 10000
E = 320000
NFEAT = 128
NHID = 128
NCLASS = 47
EPS = 1e-5

NC = 2
NS = 16
NW = NC * NS
B = 128
G = 1
GB = G * B
NG1 = 158
NG2 = 79
EPAD = NS * NG1 * GB
WH = 80
WA = 96
WB = 64
W2 = 64
RPT = 632
NACC = NS * RPT

BN = 1000
GRID = N // BN


NBUF = 6
GA = 4
SO = 2
IR = 7


def _edge_loop(ngrp, table, idxh, idxr, gbuf, acc_sh, sem, sem_i, sem_s):
  for b in range(GA + 1):
    pltpu.async_copy(idxh.at[b], idxr.at[b], sem_i)
  for b in range(GA):
    pltpu.make_async_copy(idxh.at[b], idxr.at[b], sem_i).wait()
    pltpu.async_copy(table.at[idxr.at[b, 0]], gbuf.at[b], sem)

  def grp(j, _):
    @pl.when(j >= SO)
    def _wait_scatter():
      pltpu.make_async_copy(gbuf.at[0], acc_sh.at[idxr.at[0, 1]],
                            sem_s).wait()
    nip = jnp.minimum(j + GA + 1, ngrp - 1)
    pltpu.async_copy(idxh.at[nip], idxr.at[(j + GA + 1) % IR], sem_i)
    c = jnp.minimum(j + GA, ngrp - 1)
    pltpu.make_async_copy(idxh.at[0], idxr.at[0], sem_i).wait()
    pltpu.async_copy(table.at[idxr.at[c % IR, 0]], gbuf.at[(j + GA) % NBUF],
                     sem)
    b = j % NBUF
    pltpu.make_async_copy(table.at[idxr.at[0, 0]], gbuf.at[b], sem).wait()
    pltpu.async_copy(gbuf.at[b], acc_sh.at[idxr.at[j % IR, 1]], sem_s,
                     add=True)
    return 0
  lax.fori_loop(0, ngrp, grp, 0)

  pltpu.make_async_copy(idxh.at[0], idxr.at[0], sem_i).wait()
  for _ in range(GA):
    pltpu.make_async_copy(table.at[idxr.at[0, 0]], gbuf.at[0], sem).wait()
  for _ in range(SO):
    pltpu.make_async_copy(gbuf.at[0], acc_sh.at[idxr.at[0, 1]], sem_s).wait()


def _zero_acc(gbuf, acc_sh, base, width):
  zflat = gbuf.at[0]
  zero16 = jnp.zeros((16,), jnp.float32)
  def zrow(i, _):
    for j in range(width // 16):
      zflat[i, pl.ds(j * 16, 16)] = zero16
    return 0
  lax.fori_loop(0, GB, zrow, 0)
  for k in range(RPT // GB):
    pltpu.sync_copy(zflat, acc_sh.at[pl.ds(base + k * GB, GB)])
  rem = RPT % GB
  if rem:
    pltpu.sync_copy(zflat.at[pl.ds(0, rem)],
                    acc_sh.at[pl.ds(base + (RPT // GB) * GB, rem)])


def _zero_acc_bf(gbuf, acc_sh, base, width):
  zflat = gbuf.at[0]
  zero32 = jnp.zeros((32,), jnp.bfloat16)
  def zrow(i, _):
    for j in range(width // 32):
      zflat[i, pl.ds(j * 32, 32)] = zero32
    return 0
  lax.fori_loop(0, GB, zrow, 0)
  for k in range(RPT // GB):
    pltpu.sync_copy(zflat, acc_sh.at[pl.ds(base + k * GB, GB)])
  rem = RPT % GB
  if rem:
    pltpu.sync_copy(zflat.at[pl.ds(0, rem)],
                    acc_sh.at[pl.ds(base + (RPT // GB) * GB, rem)])


def _copy_out(acc_sh, out_hbm, cid, base):
  for k in range(RPT // B):
    pltpu.sync_copy(acc_sh.at[pl.ds(base + k * B, B)],
                    out_hbm.at[cid, pl.ds(base + k * B, B)])
  rem = RPT % B
  if rem:
    pltpu.sync_copy(acc_sh.at[pl.ds(base + (RPT // B) * B, rem)],
                    out_hbm.at[cid, pl.ds(base + (RPT // B) * B, rem)])


def _copy_out2(acc_sh, out_hbm, base):
  for k in range(RPT // B):
    pltpu.sync_copy(acc_sh.at[pl.ds(base + k * B, B)],
                    out_hbm.at[pl.ds(base + k * B, B)])
  rem = RPT % B
  if rem:
    pltpu.sync_copy(acc_sh.at[pl.ds(base + (RPT // B) * B, rem)],
                    out_hbm.at[pl.ds(base + (RPT // B) * B, rem)])


_MESH = plsc.VectorSubcoreMesh(core_axis_name="c", subcore_axis_name="s")


@functools.partial(
    pl.kernel,
    out_type=(jax.ShapeDtypeStruct((NACC, WA), jnp.bfloat16),
              jax.ShapeDtypeStruct((NACC, WB), jnp.bfloat16)),
    mesh=_MESH,
    compiler_params=pltpu.CompilerParams(use_tc_tiling_on_sc=False),
    scratch_types=[
        pltpu.VMEM((IR, 2, GB), jnp.int32),
        pltpu.VMEM((NBUF, GB, WA), jnp.bfloat16),
        pltpu.VMEM((NBUF, GB, WB), jnp.bfloat16),
        pltpu.VMEM_SHARED((NACC, WA), jnp.bfloat16),
        pltpu.VMEM_SHARED((NACC, WB), jnp.bfloat16),
        pltpu.SemaphoreType.DMA,
        pltpu.SemaphoreType.DMA,
        pltpu.SemaphoreType.DMA,
    ],
)
def _sc_agg1(t0_hbm, t1_hbm, idx_hbm, o0_hbm, o1_hbm,
             idxr, gbuf0, gbuf1, acc0, acc1, sem, sem_i, sem_s):
  cid = lax.axis_index("c")
  sid = lax.axis_index("s")

  base = sid * RPT
  @pl.when(cid == 0)
  def _z0():
    _zero_acc_bf(gbuf0, acc0, base, WA)
  @pl.when(cid == 1)
  def _z1():
    _zero_acc_bf(gbuf1, acc1, base, WB)
  plsc.subcore_barrier()

  @pl.when(cid == 0)
  def _e0():
    _edge_loop(NG1, t0_hbm, idx_hbm.at[sid], idxr, gbuf0, acc0,
               sem, sem_i, sem_s)
  @pl.when(cid == 1)
  def _e1():
    _edge_loop(NG1, t1_hbm, idx_hbm.at[sid], idxr, gbuf1, acc1,
               sem, sem_i, sem_s)
  plsc.subcore_barrier()

  @pl.when(cid == 0)
  def _c0():
    _copy_out2(acc0, o0_hbm, base)
  @pl.when(cid == 1)
  def _c1():
    _copy_out2(acc1, o1_hbm, base)


@functools.partial(
    pl.kernel,
    out_type=jax.ShapeDtypeStruct((NC, NACC, W2), jnp.bfloat16),
    mesh=_MESH,
    compiler_params=pltpu.CompilerParams(use_tc_tiling_on_sc=False),
    scratch_types=[
        pltpu.VMEM((IR, 2, GB), jnp.int32),
        pltpu.VMEM((NBUF, GB, W2), jnp.bfloat16),
        pltpu.VMEM_SHARED((NACC, W2), jnp.bfloat16),
        pltpu.SemaphoreType.DMA,
        pltpu.SemaphoreType.DMA,
        pltpu.SemaphoreType.DMA,
    ],
)
def _sc_agg2(table_hbm, idx_hbm, out_hbm,
             idxr, gbuf, acc_sh, sem, sem_i, sem_s):
  cid = lax.axis_index("c")
  sid = lax.axis_index("s")
  wid = sid * NC + cid

  base = sid * RPT
  _zero_acc_bf(gbuf, acc_sh, base, W2)
  plsc.subcore_barrier()

  _edge_loop(NG2, table_hbm, idx_hbm.at[wid], idxr, gbuf, acc_sh,
             sem, sem_i, sem_s)
  plsc.subcore_barrier()

  _copy_out(acc_sh, out_hbm, cid, base)


def _tc_a_body(x_ref, wl_ref, wr_ref, b_ref, t0_ref, t1_ref, z_ref):
  xb = x_ref[...]
  y = lax.dot_general(xb, wl_ref[...], (((1,), (1,)), ((), ())),
                      preferred_element_type=jnp.float32)
  t0_ref[...] = y[:, :WA].astype(jnp.bfloat16)
  t1_ref[...] = jnp.concatenate(
      [y[:, WA:NFEAT],
       jnp.ones((BN, 16), jnp.float32),
       jnp.zeros((BN, WB - (NFEAT - WA) - 16), jnp.float32)],
      axis=1).astype(jnp.bfloat16)
  z_ref[...] = lax.dot_general(xb, wr_ref[...], (((1,), (1,)), ((), ())),
                               preferred_element_type=jnp.float32) + b_ref[...]


def _tc_b_body(p0_ref, p1_ref, z1_ref, sc_ref, sh_ref, wl_ref, wr_ref,
               b2_ref, y2_ref, z2_ref):
  p0 = p0_ref[...].astype(jnp.float32)
  p1 = p1_ref[...].astype(jnp.float32)
  feat = jnp.concatenate([p0, p1[:, :NFEAT - WA]], axis=1)
  deg = jnp.maximum(p1[:, NFEAT - WA:NFEAT - WA + 1], 1.0)
  deginv = 1.0 / deg
  h = feat * deginv + z1_ref[...]
  h = jax.nn.relu(h * sc_ref[...] + sh_ref[...])
  y2 = lax.dot_general(h, wl_ref[...], (((1,), (1,)), ((), ())),
                       preferred_element_type=jnp.float32)
  y2_ref[...] = y2.astype(jnp.bfloat16)
  z2 = lax.dot_general(h, wr_ref[...], (((1,), (1,)), ((), ())),
                       preferred_element_type=jnp.float32) + b2_ref[...]
  col = lax.broadcasted_iota(jnp.int32, (BN, W2), 1)
  z2_ref[...] = jnp.where(col == NCLASS, deginv, z2)


def _tc_c_body(p2_ref, z2_ref, o_ref):
  acc = p2_ref[0].astype(jnp.float32) + p2_ref[1].astype(jnp.float32)
  z2 = z2_ref[...]
  deginv = z2[:, NCLASS:NCLASS + 1]
  o = acc * deginv + z2
  col = lax.broadcasted_iota(jnp.int32, (BN, W2), 1)
  valid = col < NCLASS
  om = jnp.where(valid, o, -jnp.inf)
  m = jnp.max(om, axis=1, keepdims=True)
  s = jnp.sum(jnp.where(valid, jnp.exp(om - m), 0.0), axis=1, keepdims=True)
  out = o - m - jnp.log(s)
  o_ref[...] = out[:, :NCLASS]


def kernel(x, edge_index, Wl1, Wr1, b1, gamma, beta, rm, rv, Wl2, Wr2, b2):
  src = edge_index[0]
  dst = edge_index[1]
  pad = EPAD - E
  srcp = jnp.concatenate([src, jnp.zeros((pad,), jnp.int32)])
  dstp = jnp.concatenate([dst, jnp.full((pad,), N, jnp.int32)])
  idx1 = jnp.stack([srcp.reshape(NS, NG1, GB),
                    dstp.reshape(NS, NG1, GB)], axis=2)
  idx2 = jnp.stack([srcp.reshape(NW, NG2, GB),
                    dstp.reshape(NW, NG2, GB)], axis=2)

  bn_mul = gamma * lax.rsqrt(rv + EPS)
  bn_scale = bn_mul.reshape(1, NHID)
  bn_shift = (beta - rm * bn_mul).reshape(1, NHID)
  b1r = b1.reshape(1, NHID)
  Wl2p = jnp.zeros((W2, NHID), jnp.float32).at[:NCLASS].set(Wl2)
  Wr2p = jnp.zeros((W2, NHID), jnp.float32).at[:NCLASS].set(Wr2)
  b2p = jnp.zeros((1, W2), jnp.float32).at[0, :NCLASS].set(b2)
  del gamma, beta, rm, rv

  full = lambda shape: pl.BlockSpec(shape, lambda i: (0,) * len(shape))

  t0, t1, z1 = pl.pallas_call(
      _tc_a_body,
      grid=(GRID,),
      in_specs=[
          pl.BlockSpec((BN, NFEAT), lambda i: (i, 0)),
          full((NHID, NFEAT)), full((NHID, NFEAT)), full((1, NHID)),
      ],
      out_specs=[
          pl.BlockSpec((BN, WA), lambda i: (i, 0)),
          pl.BlockSpec((BN, WB), lambda i: (i, 0)),
          pl.BlockSpec((BN, NHID), lambda i: (i, 0)),
      ],
      out_shape=[
          jax.ShapeDtypeStruct((N, WA), jnp.bfloat16),
          jax.ShapeDtypeStruct((N, WB), jnp.bfloat16),
          jax.ShapeDtypeStruct((N, NHID), jnp.float32),
      ],
  )(x, Wl1, Wr1, b1r)

  p0, p1 = _sc_agg1(t0, t1, idx1)

  y2, z2 = pl.pallas_call(
      _tc_b_body,
      grid=(GRID,),
      in_specs=[
          pl.BlockSpec((BN, WA), lambda i: (i, 0)),
          pl.BlockSpec((BN, WB), lambda i: (i, 0)),
          pl.BlockSpec((BN, NHID), lambda i: (i, 0)),
          full((1, NHID)), full((1, NHID)),
          full((W2, NHID)), full((W2, NHID)), full((1, W2)),
      ],
      out_specs=[
          pl.BlockSpec((BN, W2), lambda i: (i, 0)),
          pl.BlockSpec((BN, W2), lambda i: (i, 0)),
      ],
      out_shape=[
          jax.ShapeDtypeStruct((N, W2), jnp.bfloat16),
          jax.ShapeDtypeStruct((N, W2), jnp.float32),
      ],
  )(p0, p1, z1, bn_scale, bn_shift, Wl2p, Wr2p, b2p)

  p2 = _sc_agg2(y2, idx2)

  out = pl.pallas_call(
      _tc_c_body,
      grid=(GRID,),
      in_specs=[
          pl.BlockSpec((NC, BN, W2), lambda i: (0, i, 0)),
          pl.BlockSpec((BN, W2), lambda i: (i, 0)),
      ],
      out_specs=pl.BlockSpec((BN, NCLASS), lambda i: (i, 0)),
      out_shape=jax.ShapeDtypeStruct((N, NCLASS), jnp.float32),
  )(p2, z2)
  return out


# speedup vs baseline: 18.3948x; 1.5491x over previous
import functools

import jax
import jax.numpy as jnp
from jax import lax
from jax.experimental import pallas as pl
from jax.experimental.pallas import tpu as pltpu
from jax.experimental.pallas import tpu_sc as plsc

N = 10000
E = 320000
NFEAT = 128
NHID = 128
NCLASS = 47
EPS = 1e-5

NC = 2
NS = 16
NW = NC * NS
B = 128
GB = B
NCH = E // B
W1 = 64
WD = 32
W2 = 64
RPT = 632
NACC = NS * RPT

BN = 1000
GRID = N // BN


NBUF = 8
GA = 6
SO = 2
IR = 9


def _edge_loop(ngrp, lo, table, eidx, idxr, gbuf, acc_sh, sem, sem_i,
               sem_s, deg=None):
  def _pf(c, slot):
    off = (lo + c) * B
    pltpu.async_copy(eidx.at[0, pl.ds(off, B)], idxr.at[slot, 0], sem_i)
    pltpu.async_copy(eidx.at[1, pl.ds(off, B)], idxr.at[slot, 1], sem_i)

  def _pf_wait():
    pltpu.make_async_copy(eidx.at[0, pl.ds(0, B)], idxr.at[0, 0],
                          sem_i).wait()
    pltpu.make_async_copy(eidx.at[0, pl.ds(0, B)], idxr.at[0, 0],
                          sem_i).wait()

  for b in range(GA + 1):
    _pf(b, b)
  for b in range(GA):
    _pf_wait()
    pltpu.async_copy(table.at[idxr.at[b, 0]], gbuf.at[b], sem)

  def grp(j, _):
    @pl.when(j >= SO)
    def _wait_scatter():
      pltpu.make_async_copy(gbuf.at[0], acc_sh.at[idxr.at[0, 1]],
                            sem_s).wait()
      if deg is not None:
        pltpu.make_async_copy(deg[0], deg[1].at[idxr.at[0, 1]],
                              deg[2]).wait()
    _pf(jnp.minimum(j + GA + 1, ngrp - 1), (j + GA + 1) % IR)
    c = jnp.minimum(j + GA, ngrp - 1)
    _pf_wait()
    pltpu.async_copy(table.at[idxr.at[c % IR, 0]], gbuf.at[(j + GA) % NBUF],
                     sem)
    b = j % NBUF
    pltpu.make_async_copy(table.at[idxr.at[0, 0]], gbuf.at[b], sem).wait()
    pltpu.async_copy(gbuf.at[b], acc_sh.at[idxr.at[j % IR, 1]], sem_s,
                     add=True)
    if deg is not None:
      pltpu.async_copy(deg[0], deg[1].at[idxr.at[j % IR, 1]], deg[2],
                       add=True)
    return 0
  lax.fori_loop(0, ngrp, grp, 0)

  _pf_wait()
  for _ in range(GA):
    pltpu.make_async_copy(table.at[idxr.at[0, 0]], gbuf.at[0], sem).wait()
  for _ in range(SO):
    pltpu.make_async_copy(gbuf.at[0], acc_sh.at[idxr.at[0, 1]], sem_s).wait()
    if deg is not None:
      pltpu.make_async_copy(deg[0], deg[1].at[idxr.at[0, 1]], deg[2]).wait()


def _zero_acc(gbuf, acc_sh, base, width):
  zflat = gbuf.at[0]
  zero16 = jnp.zeros((16,), jnp.float32)
  def zrow(i, _):
    for j in range(width // 16):
      zflat[i, pl.ds(j * 16, 16)] = zero16
    return 0
  lax.fori_loop(0, GB, zrow, 0)
  for k in range(RPT // GB):
    pltpu.sync_copy(zflat, acc_sh.at[pl.ds(base + k * GB, GB)])
  rem = RPT % GB
  if rem:
    pltpu.sync_copy(zflat.at[pl.ds(0, rem)],
                    acc_sh.at[pl.ds(base + (RPT // GB) * GB, rem)])


def _zero_acc_bf(gbuf, acc_sh, base, width):
  zflat = gbuf.at[0]
  zero32 = jnp.zeros((32,), jnp.bfloat16)
  def zrow(i, _):
    for j in range(width // 32):
      zflat[i, pl.ds(j * 32, 32)] = zero32
    return 0
  lax.fori_loop(0, GB, zrow, 0)
  for k in range(RPT // GB):
    pltpu.sync_copy(zflat, acc_sh.at[pl.ds(base + k * GB, GB)])
  rem = RPT % GB
  if rem:
    pltpu.sync_copy(zflat.at[pl.ds(0, rem)],
                    acc_sh.at[pl.ds(base + (RPT // GB) * GB, rem)])


def _copy_out(acc_sh, out_hbm, cid, base):
  for k in range(RPT // B):
    pltpu.sync_copy(acc_sh.at[pl.ds(base + k * B, B)],
                    out_hbm.at[cid, pl.ds(base + k * B, B)])
  rem = RPT % B
  if rem:
    pltpu.sync_copy(acc_sh.at[pl.ds(base + (RPT // B) * B, rem)],
                    out_hbm.at[cid, pl.ds(base + (RPT // B) * B, rem)])


def _copy_out2(acc_sh, out_hbm, base):
  for k in range(RPT // B):
    pltpu.sync_copy(acc_sh.at[pl.ds(base + k * B, B)],
                    out_hbm.at[pl.ds(base + k * B, B)])
  rem = RPT % B
  if rem:
    pltpu.sync_copy(acc_sh.at[pl.ds(base + (RPT // B) * B, rem)],
                    out_hbm.at[pl.ds(base + (RPT // B) * B, rem)])


_MESH = plsc.VectorSubcoreMesh(core_axis_name="c", subcore_axis_name="s")


@functools.partial(
    pl.kernel,
    out_type=(jax.ShapeDtypeStruct((NC, NACC, W1), jnp.bfloat16),
              jax.ShapeDtypeStruct((NACC, WD), jnp.bfloat16)),
    mesh=_MESH,
    compiler_params=pltpu.CompilerParams(use_tc_tiling_on_sc=False),
    scratch_types=[
        pltpu.VMEM((IR, 2, GB), jnp.int32),
        pltpu.VMEM((NBUF, GB, W1), jnp.bfloat16),
        pltpu.VMEM((GB, WD), jnp.bfloat16),
        pltpu.VMEM((GB, WD), jnp.bfloat16),
        pltpu.VMEM_SHARED((NACC, W1), jnp.bfloat16),
        pltpu.VMEM_SHARED((NACC, WD), jnp.bfloat16),
        pltpu.SemaphoreType.DMA,
        pltpu.SemaphoreType.DMA,
        pltpu.SemaphoreType.DMA,
        pltpu.SemaphoreType.DMA,
    ],
)
def _sc_agg1(t_hbm, eidx_hbm, o_hbm, od_hbm,
             idxr, gbuf, ones_v, zd_v, acc, acc_deg, sem, sem_i, sem_s,
             sem_d):
  cid = lax.axis_index("c")
  sid = lax.axis_index("s")

  base = sid * RPT
  _zero_acc_bf(gbuf, acc, base, W1)
  one32 = jnp.ones((32,), jnp.bfloat16)
  def orow(i, _):
    ones_v[i, pl.ds(0, 32)] = one32
    return 0
  lax.fori_loop(0, GB, orow, 0)

  @pl.when(cid == 1)
  def _zd():
    zero32 = jnp.zeros((32,), jnp.bfloat16)
    def zrow2(i, _):
      zd_v[i, pl.ds(0, 32)] = zero32
      return 0
    lax.fori_loop(0, GB, zrow2, 0)
    for k in range(RPT // GB):
      pltpu.sync_copy(zd_v, acc_deg.at[pl.ds(base + k * GB, GB)])
    rem = RPT % GB
    if rem:
      pltpu.sync_copy(zd_v.at[pl.ds(0, rem)],
                      acc_deg.at[pl.ds(base + (RPT // GB) * GB, rem)])
  plsc.subcore_barrier()

  ngrp = jnp.where(sid < NCH % NS, NCH // NS + 1, NCH // NS)
  lo = sid * (NCH // NS) + jnp.minimum(sid, NCH % NS)
  @pl.when(cid == 0)
  def _e0():
    _edge_loop(ngrp, lo, t_hbm.at[0], eidx_hbm, idxr, gbuf, acc,
               sem, sem_i, sem_s)
  @pl.when(cid == 1)
  def _e1():
    _edge_loop(ngrp, lo, t_hbm.at[1], eidx_hbm, idxr, gbuf, acc,
               sem, sem_i, sem_s, deg=(ones_v, acc_deg, sem_d))
  plsc.subcore_barrier()

  _copy_out(acc, o_hbm, cid, base)
  @pl.when(cid == 1)
  def _cd():
    _copy_out2(acc_deg, od_hbm, base)


@functools.partial(
    pl.kernel,
    out_type=jax.ShapeDtypeStruct((NC, NACC, W2), jnp.bfloat16),
    mesh=_MESH,
    compiler_params=pltpu.CompilerParams(use_tc_tiling_on_sc=False),
    scratch_types=[
        pltpu.VMEM((IR, 2, GB), jnp.int32),
        pltpu.VMEM((NBUF, GB, W2), jnp.bfloat16),
        pltpu.VMEM_SHARED((NACC, W2), jnp.bfloat16),
        pltpu.SemaphoreType.DMA,
        pltpu.SemaphoreType.DMA,
        pltpu.SemaphoreType.DMA,
    ],
)
def _sc_agg2(table_hbm, eidx_hbm, out_hbm,
             idxr, gbuf, acc_sh, sem, sem_i, sem_s):
  cid = lax.axis_index("c")
  sid = lax.axis_index("s")
  wid = sid * NC + cid

  base = sid * RPT
  _zero_acc_bf(gbuf, acc_sh, base, W2)
  plsc.subcore_barrier()

  ngrp = jnp.where(wid < NCH % NW, NCH // NW + 1, NCH // NW)
  lo = wid * (NCH // NW) + jnp.minimum(wid, NCH % NW)
  _edge_loop(ngrp, lo, table_hbm, eidx_hbm, idxr, gbuf, acc_sh,
             sem, sem_i, sem_s)
  plsc.subcore_barrier()

  _copy_out(acc_sh, out_hbm, cid, base)


def _tc_a_body(x_ref, wl_ref, wr_ref, b_ref, t_ref, z_ref):
  xb = x_ref[...]
  y = lax.dot_general(xb, wl_ref[...], (((1,), (1,)), ((), ())),
                      preferred_element_type=jnp.float32)
  t_ref[0] = y[:, :W1].astype(jnp.bfloat16)
  t_ref[1] = y[:, W1:].astype(jnp.bfloat16)
  z_ref[...] = lax.dot_general(xb, wr_ref[...], (((1,), (1,)), ((), ())),
                               preferred_element_type=jnp.float32) + b_ref[...]


def _tc_b_body(p_ref, pd_ref, z1_ref, sc_ref, sh_ref, wl_ref, wr_ref,
               b2_ref, y2_ref, z2_ref):
  feat = jnp.concatenate([p_ref[0].astype(jnp.float32),
                          p_ref[1].astype(jnp.float32)], axis=1)
  deg = jnp.maximum(pd_ref[...][:, 0:1].astype(jnp.float32), 1.0)
  deginv = 1.0 / deg
  h = feat * deginv + z1_ref[...]
  h = jax.nn.relu(h * sc_ref[...] + sh_ref[...])
  y2 = lax.dot_general(h, wl_ref[...], (((1,), (1,)), ((), ())),
                       preferred_element_type=jnp.float32)
  y2_ref[...] = y2.astype(jnp.bfloat16)
  z2 = lax.dot_general(h, wr_ref[...], (((1,), (1,)), ((), ())),
                       preferred_element_type=jnp.float32) + b2_ref[...]
  col = lax.broadcasted_iota(jnp.int32, (BN, W2), 1)
  z2_ref[...] = jnp.where(col == NCLASS, deginv, z2)


def _tc_c_body(p2_ref, z2_ref, o_ref):
  acc = p2_ref[0].astype(jnp.float32) + p2_ref[1].astype(jnp.float32)
  z2 = z2_ref[...]
  deginv = z2[:, NCLASS:NCLASS + 1]
  o = acc * deginv + z2
  col = lax.broadcasted_iota(jnp.int32, (BN, W2), 1)
  valid = col < NCLASS
  om = jnp.where(valid, o, -jnp.inf)
  m = jnp.max(om, axis=1, keepdims=True)
  s = jnp.sum(jnp.where(valid, jnp.exp(om - m), 0.0), axis=1, keepdims=True)
  out = o - m - jnp.log(s)
  o_ref[...] = out[:, :NCLASS]


def kernel(x, edge_index, Wl1, Wr1, b1, gamma, beta, rm, rv, Wl2, Wr2, b2):
  bn_mul = gamma * lax.rsqrt(rv + EPS)
  bn_scale = bn_mul.reshape(1, NHID)
  bn_shift = (beta - rm * bn_mul).reshape(1, NHID)
  b1r = b1.reshape(1, NHID)
  Wl2p = jnp.zeros((W2, NHID), jnp.float32).at[:NCLASS].set(Wl2)
  Wr2p = jnp.zeros((W2, NHID), jnp.float32).at[:NCLASS].set(Wr2)
  b2p = jnp.zeros((1, W2), jnp.float32).at[0, :NCLASS].set(b2)
  del gamma, beta, rm, rv

  full = lambda shape: pl.BlockSpec(shape, lambda i: (0,) * len(shape))

  t1tab, z1 = pl.pallas_call(
      _tc_a_body,
      grid=(GRID,),
      in_specs=[
          pl.BlockSpec((BN, NFEAT), lambda i: (i, 0)),
          full((NHID, NFEAT)), full((NHID, NFEAT)), full((1, NHID)),
      ],
      out_specs=[
          pl.BlockSpec((NC, BN, W1), lambda i: (0, i, 0)),
          pl.BlockSpec((BN, NHID), lambda i: (i, 0)),
      ],
      out_shape=[
          jax.ShapeDtypeStruct((NC, N, W1), jnp.bfloat16),
          jax.ShapeDtypeStruct((N, NHID), jnp.float32),
      ],
  )(x, Wl1, Wr1, b1r)

  p1, pdeg = _sc_agg1(t1tab, edge_index)

  y2, z2 = pl.pallas_call(
      _tc_b_body,
      grid=(GRID,),
      in_specs=[
          pl.BlockSpec((NC, BN, W1), lambda i: (0, i, 0)),
          pl.BlockSpec((BN, WD), lambda i: (i, 0)),
          pl.BlockSpec((BN, NHID), lambda i: (i, 0)),
          full((1, NHID)), full((1, NHID)),
          full((W2, NHID)), full((W2, NHID)), full((1, W2)),
      ],
      out_specs=[
          pl.BlockSpec((BN, W2), lambda i: (i, 0)),
          pl.BlockSpec((BN, W2), lambda i: (i, 0)),
      ],
      out_shape=[
          jax.ShapeDtypeStruct((N, W2), jnp.bfloat16),
          jax.ShapeDtypeStruct((N, W2), jnp.float32),
      ],
  )(p1, pdeg, z1, bn_scale, bn_shift, Wl2p, Wr2p, b2p)

  p2 = _sc_agg2(y2, edge_index)

  out = pl.pallas_call(
      _tc_c_body,
      grid=(GRID,),
      in_specs=[
          pl.BlockSpec((NC, BN, W2), lambda i: (0, i, 0)),
          pl.BlockSpec((BN, W2), lambda i: (i, 0)),
      ],
      out_specs=pl.BlockSpec((BN, NCLASS), lambda i: (i, 0)),
      out_shape=jax.ShapeDtypeStruct((N, NCLASS), jnp.float32),
  )(p2, z2)
  return out


# speedup vs baseline: 18.4458x; 1.0028x over previous
import functools

import jax
import jax.numpy as jnp
from jax import lax
from jax.experimental import pallas as pl
from jax.experimental.pallas import tpu as pltpu
from jax.experimental.pallas import tpu_sc as plsc

N = 10000
E = 320000
NFEAT = 128
NHID = 128
NCLASS = 47
EPS = 1e-5

NC = 2
NS = 16
NW = NC * NS
B = 128
GB = B
NCH = E // B
W1 = 64
WD = 32
W2 = 64
RPT = 632
NACC = NS * RPT

BN = 1000
GRID = N // BN


NBUF = 12
GA = 6
SO = 2
IR = 15


def _edge_loop(ngrp, lo, table, eidx, idxr, gbuf, acc_sh, sem, sem_i,
               sem_s, deg=None):
  def _pf(c, slot):
    off = (lo + c) * B
    pltpu.async_copy(eidx.at[0, pl.ds(off, B)], idxr.at[slot, 0], sem_i)
    pltpu.async_copy(eidx.at[1, pl.ds(off, B)], idxr.at[slot, 1], sem_i)

  def _pf_wait():
    pltpu.make_async_copy(eidx.at[0, pl.ds(0, B)], idxr.at[0, 0],
                          sem_i).wait()
    pltpu.make_async_copy(eidx.at[0, pl.ds(0, B)], idxr.at[0, 0],
                          sem_i).wait()

  for b in range(GA + 1):
    _pf(b, b)
  for b in range(GA):
    _pf_wait()
    pltpu.async_copy(table.at[idxr.at[b, 0]], gbuf.at[b], sem)

  def grp(j, _):
    @pl.when(j >= SO)
    def _wait_scatter():
      pltpu.make_async_copy(gbuf.at[0], acc_sh.at[idxr.at[0, 1]],
                            sem_s).wait()
      if deg is not None:
        pltpu.make_async_copy(deg[0], deg[1].at[idxr.at[0, 1]],
                              deg[2]).wait()
    _pf(jnp.minimum(j + GA + 1, ngrp - 1), (j + GA + 1) % IR)
    c = jnp.minimum(j + GA, ngrp - 1)
    _pf_wait()
    pltpu.async_copy(table.at[idxr.at[c % IR, 0]], gbuf.at[(j + GA) % NBUF],
                     sem)
    b = j % NBUF
    pltpu.make_async_copy(table.at[idxr.at[0, 0]], gbuf.at[b], sem).wait()
    pltpu.async_copy(gbuf.at[b], acc_sh.at[idxr.at[j % IR, 1]], sem_s,
                     add=True)
    if deg is not None:
      pltpu.async_copy(deg[0], deg[1].at[idxr.at[j % IR, 1]], deg[2],
                       add=True)
    return 0
  lax.fori_loop(0, ngrp, grp, 0)

  _pf_wait()
  for _ in range(GA):
    pltpu.make_async_copy(table.at[idxr.at[0, 0]], gbuf.at[0], sem).wait()
  for _ in range(SO):
    pltpu.make_async_copy(gbuf.at[0], acc_sh.at[idxr.at[0, 1]], sem_s).wait()
    if deg is not None:
      pltpu.make_async_copy(deg[0], deg[1].at[idxr.at[0, 1]], deg[2]).wait()


def _zero_acc(gbuf, acc_sh, base, width):
  zflat = gbuf.at[0]
  zero16 = jnp.zeros((16,), jnp.float32)
  def zrow(i, _):
    for j in range(width // 16):
      zflat[i, pl.ds(j * 16, 16)] = zero16
    return 0
  lax.fori_loop(0, GB, zrow, 0)
  for k in range(RPT // GB):
    pltpu.sync_copy(zflat, acc_sh.at[pl.ds(base + k * GB, GB)])
  rem = RPT % GB
  if rem:
    pltpu.sync_copy(zflat.at[pl.ds(0, rem)],
                    acc_sh.at[pl.ds(base + (RPT // GB) * GB, rem)])


def _zero_acc_bf(gbuf, acc_sh, base, width):
  zflat = gbuf.at[0]
  zero32 = jnp.zeros((32,), jnp.bfloat16)
  def zrow(i, _):
    for j in range(width // 32):
      zflat[i, pl.ds(j * 32, 32)] = zero32
    return 0
  lax.fori_loop(0, GB, zrow, 0)
  for k in range(RPT // GB):
    pltpu.sync_copy(zflat, acc_sh.at[pl.ds(base + k * GB, GB)])
  rem = RPT % GB
  if rem:
    pltpu.sync_copy(zflat.at[pl.ds(0, rem)],
                    acc_sh.at[pl.ds(base + (RPT // GB) * GB, rem)])


def _copy_out(acc_sh, out_hbm, cid, base):
  for k in range(RPT // B):
    pltpu.sync_copy(acc_sh.at[pl.ds(base + k * B, B)],
                    out_hbm.at[cid, pl.ds(base + k * B, B)])
  rem = RPT % B
  if rem:
    pltpu.sync_copy(acc_sh.at[pl.ds(base + (RPT // B) * B, rem)],
                    out_hbm.at[cid, pl.ds(base + (RPT // B) * B, rem)])


def _copy_out2(acc_sh, out_hbm, base):
  for k in range(RPT // B):
    pltpu.sync_copy(acc_sh.at[pl.ds(base + k * B, B)],
                    out_hbm.at[pl.ds(base + k * B, B)])
  rem = RPT % B
  if rem:
    pltpu.sync_copy(acc_sh.at[pl.ds(base + (RPT // B) * B, rem)],
                    out_hbm.at[pl.ds(base + (RPT // B) * B, rem)])


_MESH = plsc.VectorSubcoreMesh(core_axis_name="c", subcore_axis_name="s")


@functools.partial(
    pl.kernel,
    out_type=(jax.ShapeDtypeStruct((NC, NACC, W1), jnp.bfloat16),
              jax.ShapeDtypeStruct((NACC, WD), jnp.bfloat16)),
    mesh=_MESH,
    compiler_params=pltpu.CompilerParams(use_tc_tiling_on_sc=False),
    scratch_types=[
        pltpu.VMEM((IR, 2, GB), jnp.int32),
        pltpu.VMEM((NBUF, GB, W1), jnp.bfloat16),
        pltpu.VMEM((GB, WD), jnp.bfloat16),
        pltpu.VMEM((GB, WD), jnp.bfloat16),
        pltpu.VMEM_SHARED((NACC, W1), jnp.bfloat16),
        pltpu.VMEM_SHARED((NACC, WD), jnp.bfloat16),
        pltpu.SemaphoreType.DMA,
        pltpu.SemaphoreType.DMA,
        pltpu.SemaphoreType.DMA,
        pltpu.SemaphoreType.DMA,
    ],
)
def _sc_agg1(t_hbm, eidx_hbm, o_hbm, od_hbm,
             idxr, gbuf, ones_v, zd_v, acc, acc_deg, sem, sem_i, sem_s,
             sem_d):
  cid = lax.axis_index("c")
  sid = lax.axis_index("s")

  base = sid * RPT
  _zero_acc_bf(gbuf, acc, base, W1)
  one32 = jnp.ones((32,), jnp.bfloat16)
  def orow(i, _):
    ones_v[i, pl.ds(0, 32)] = one32
    return 0
  lax.fori_loop(0, GB, orow, 0)

  @pl.when(cid == 1)
  def _zd():
    zero32 = jnp.zeros((32,), jnp.bfloat16)
    def zrow2(i, _):
      zd_v[i, pl.ds(0, 32)] = zero32
      return 0
    lax.fori_loop(0, GB, zrow2, 0)
    for k in range(RPT // GB):
      pltpu.sync_copy(zd_v, acc_deg.at[pl.ds(base + k * GB, GB)])
    rem = RPT % GB
    if rem:
      pltpu.sync_copy(zd_v.at[pl.ds(0, rem)],
                      acc_deg.at[pl.ds(base + (RPT // GB) * GB, rem)])
  plsc.subcore_barrier()

  ngrp = jnp.where(sid < NCH % NS, NCH // NS + 1, NCH // NS)
  lo = sid * (NCH // NS) + jnp.minimum(sid, NCH % NS)
  @pl.when(cid == 0)
  def _e0():
    _edge_loop(ngrp, lo, t_hbm.at[0], eidx_hbm, idxr, gbuf, acc,
               sem, sem_i, sem_s)
  @pl.when(cid == 1)
  def _e1():
    _edge_loop(ngrp, lo, t_hbm.at[1], eidx_hbm, idxr, gbuf, acc,
               sem, sem_i, sem_s, deg=(ones_v, acc_deg, sem_d))
  plsc.subcore_barrier()

  _copy_out(acc, o_hbm, cid, base)
  @pl.when(cid == 1)
  def _cd():
    _copy_out2(acc_deg, od_hbm, base)


@functools.partial(
    pl.kernel,
    out_type=jax.ShapeDtypeStruct((NC, NACC, W2), jnp.bfloat16),
    mesh=_MESH,
    compiler_params=pltpu.CompilerParams(use_tc_tiling_on_sc=False),
    scratch_types=[
        pltpu.VMEM((IR, 2, GB), jnp.int32),
        pltpu.VMEM((NBUF, GB, W2), jnp.bfloat16),
        pltpu.VMEM_SHARED((NACC, W2), jnp.bfloat16),
        pltpu.SemaphoreType.DMA,
        pltpu.SemaphoreType.DMA,
        pltpu.SemaphoreType.DMA,
    ],
)
def _sc_agg2(table_hbm, eidx_hbm, out_hbm,
             idxr, gbuf, acc_sh, sem, sem_i, sem_s):
  cid = lax.axis_index("c")
  sid = lax.axis_index("s")
  wid = sid * NC + cid

  base = sid * RPT
  _zero_acc_bf(gbuf, acc_sh, base, W2)
  plsc.subcore_barrier()

  ngrp = jnp.where(wid < NCH % NW, NCH // NW + 1, NCH // NW)
  lo = wid * (NCH // NW) + jnp.minimum(wid, NCH % NW)
  _edge_loop(ngrp, lo, table_hbm, eidx_hbm, idxr, gbuf, acc_sh,
             sem, sem_i, sem_s)
  plsc.subcore_barrier()

  _copy_out(acc_sh, out_hbm, cid, base)


def _tc_a_body(x_ref, wl_ref, wr_ref, b_ref, t_ref, z_ref):
  xb = x_ref[...]
  y = lax.dot_general(xb, wl_ref[...], (((1,), (1,)), ((), ())),
                      preferred_element_type=jnp.float32)
  t_ref[0] = y[:, :W1].astype(jnp.bfloat16)
  t_ref[1] = y[:, W1:].astype(jnp.bfloat16)
  z_ref[...] = lax.dot_general(xb, wr_ref[...], (((1,), (1,)), ((), ())),
                               preferred_element_type=jnp.float32) + b_ref[...]


def _tc_b_body(p_ref, pd_ref, z1_ref, sc_ref, sh_ref, wl_ref, wr_ref,
               b2_ref, y2_ref, z2_ref):
  feat = jnp.concatenate([p_ref[0].astype(jnp.float32),
                          p_ref[1].astype(jnp.float32)], axis=1)
  deg = jnp.maximum(pd_ref[...][:, 0:1].astype(jnp.float32), 1.0)
  deginv = 1.0 / deg
  h = feat * deginv + z1_ref[...]
  h = jax.nn.relu(h * sc_ref[...] + sh_ref[...])
  y2 = lax.dot_general(h, wl_ref[...], (((1,), (1,)), ((), ())),
                       preferred_element_type=jnp.float32)
  y2_ref[...] = y2.astype(jnp.bfloat16)
  z2 = lax.dot_general(h, wr_ref[...], (((1,), (1,)), ((), ())),
                       preferred_element_type=jnp.float32) + b2_ref[...]
  col = lax.broadcasted_iota(jnp.int32, (BN, W2), 1)
  z2_ref[...] = jnp.where(col == NCLASS, deginv, z2)


def _tc_c_body(p2_ref, z2_ref, o_ref):
  acc = p2_ref[0].astype(jnp.float32) + p2_ref[1].astype(jnp.float32)
  z2 = z2_ref[...]
  deginv = z2[:, NCLASS:NCLASS + 1]
  o = acc * deginv + z2
  col = lax.broadcasted_iota(jnp.int32, (BN, W2), 1)
  valid = col < NCLASS
  om = jnp.where(valid, o, -jnp.inf)
  m = jnp.max(om, axis=1, keepdims=True)
  s = jnp.sum(jnp.where(valid, jnp.exp(om - m), 0.0), axis=1, keepdims=True)
  out = o - m - jnp.log(s)
  o_ref[...] = out[:, :NCLASS]


def kernel(x, edge_index, Wl1, Wr1, b1, gamma, beta, rm, rv, Wl2, Wr2, b2):
  bn_mul = gamma * lax.rsqrt(rv + EPS)
  bn_scale = bn_mul.reshape(1, NHID)
  bn_shift = (beta - rm * bn_mul).reshape(1, NHID)
  b1r = b1.reshape(1, NHID)
  Wl2p = jnp.zeros((W2, NHID), jnp.float32).at[:NCLASS].set(Wl2)
  Wr2p = jnp.zeros((W2, NHID), jnp.float32).at[:NCLASS].set(Wr2)
  b2p = jnp.zeros((1, W2), jnp.float32).at[0, :NCLASS].set(b2)
  del gamma, beta, rm, rv

  full = lambda shape: pl.BlockSpec(shape, lambda i: (0,) * len(shape))

  t1tab, z1 = pl.pallas_call(
      _tc_a_body,
      grid=(GRID,),
      in_specs=[
          pl.BlockSpec((BN, NFEAT), lambda i: (i, 0)),
          full((NHID, NFEAT)), full((NHID, NFEAT)), full((1, NHID)),
      ],
      out_specs=[
          pl.BlockSpec((NC, BN, W1), lambda i: (0, i, 0)),
          pl.BlockSpec((BN, NHID), lambda i: (i, 0)),
      ],
      out_shape=[
          jax.ShapeDtypeStruct((NC, N, W1), jnp.bfloat16),
          jax.ShapeDtypeStruct((N, NHID), jnp.float32),
      ],
  )(x, Wl1, Wr1, b1r)

  p1, pdeg = _sc_agg1(t1tab, edge_index)

  y2, z2 = pl.pallas_call(
      _tc_b_body,
      grid=(GRID,),
      in_specs=[
          pl.BlockSpec((NC, BN, W1), lambda i: (0, i, 0)),
          pl.BlockSpec((BN, WD), lambda i: (i, 0)),
          pl.BlockSpec((BN, NHID), lambda i: (i, 0)),
          full((1, NHID)), full((1, NHID)),
          full((W2, NHID)), full((W2, NHID)), full((1, W2)),
      ],
      out_specs=[
          pl.BlockSpec((BN, W2), lambda i: (i, 0)),
          pl.BlockSpec((BN, W2), lambda i: (i, 0)),
      ],
      out_shape=[
          jax.ShapeDtypeStruct((N, W2), jnp.bfloat16),
          jax.ShapeDtypeStruct((N, W2), jnp.float32),
      ],
  )(p1, pdeg, z1, bn_scale, bn_shift, Wl2p, Wr2p, b2p)

  p2 = _sc_agg2(y2, edge_index)

  out = pl.pallas_call(
      _tc_c_body,
      grid=(GRID,),
      in_specs=[
          pl.BlockSpec((NC, BN, W2), lambda i: (0, i, 0)),
          pl.BlockSpec((BN, W2), lambda i: (i, 0)),
      ],
      out_specs=pl.BlockSpec((BN, NCLASS), lambda i: (i, 0)),
      out_shape=jax.ShapeDtypeStruct((N, NCLASS), jnp.float32),
  )(p2, z2)
  return out
